# initial kernel scaffold (unmeasured)
import jax
import jax.numpy as jnp
from jax import lax
from jax.experimental import pallas as pl
from jax.experimental.pallas import tpu as pltpu


def kernel(
    t,
):
    def body(*refs):
        pass

    out_shape = jax.ShapeDtypeStruct(..., jnp.float32)
    return pl.pallas_call(body, out_shape=out_shape)(...)


# baseline (device time: 1194469 ns/iter reference)
import jax
import jax.numpy as jnp
from jax import lax
from jax.experimental import pallas as pl
from jax.experimental.pallas import tpu as pltpu

N_DEV = 4
NPASS = 2


def _f(s):
    r = jnp.maximum(s, 0.0)
    return jnp.tanh(s) * s * s + r * r * r


def kernel(t):
    m, n = t.shape
    c = m // N_DEV
    r_rows = c // NPASS

    def body(t_ref, out_ref, comm_ref, acc_ref, local_ref,
             send_sems, recv_sems, load_sem, store_sem, hop_sem):
        my = lax.axis_index("i")
        left = (my + N_DEV - 1) % N_DEV
        right = (my + 1) % N_DEV

        def nbr_barrier(sem):
            for nbr in (left, right):
                pl.semaphore_signal(
                    sem, inc=1,
                    device_id=(nbr,), device_id_type=pl.DeviceIdType.MESH,
                )
            pl.semaphore_wait(sem, 2)

        nbr_barrier(pltpu.get_barrier_semaphore())

        for p in range(NPASS):
            def rows(chunk_idx):
                return pl.ds(chunk_idx * c + p * r_rows, r_rows)

            cp = pltpu.make_async_copy(t_ref.at[rows(my), :], acc_ref, load_sem)
            cp.start()
            cp.wait()

            for s in range(N_DEV - 1):
                h = s
                if not (p == 0 and h == 0):
                    nbr_barrier(hop_sem)
                slot = h % 2
                rdma = pltpu.make_async_remote_copy(
                    src_ref=acc_ref,
                    dst_ref=comm_ref.at[slot],
                    send_sem=send_sems.at[h],
                    recv_sem=recv_sems.at[h],
                    device_id=(right,),
                    device_id_type=pl.DeviceIdType.MESH,
                )
                rdma.start()
                incoming = (my + N_DEV - s - 1) % N_DEV
                load = pltpu.make_async_copy(
                    t_ref.at[rows(incoming), :], local_ref, load_sem
                )
                load.start()
                rdma.wait()
                load.wait()
                acc_ref[...] = comm_ref[slot] + local_ref[...]

            own = (my + 1) % N_DEV
            acc_ref[...] = _f(acc_ref[...])
            st = pltpu.make_async_copy(acc_ref, out_ref.at[rows(own), :], store_sem)
            st.start()
            st.wait()

            for s in range(N_DEV - 1):
                h = N_DEV - 1 + s
                nbr_barrier(hop_sem)
                slot = h % 2
                src = acc_ref if s == 0 else comm_ref.at[(h - 1) % 2]
                rdma = pltpu.make_async_remote_copy(
                    src_ref=src,
                    dst_ref=comm_ref.at[slot],
                    send_sem=send_sems.at[h],
                    recv_sem=recv_sems.at[h],
                    device_id=(right,),
                    device_id_type=pl.DeviceIdType.MESH,
                )
                rdma.start()
                rdma.wait()
                got = (my + N_DEV - s) % N_DEV
                st = pltpu.make_async_copy(
                    comm_ref.at[slot], out_ref.at[rows(got), :], store_sem
                )
                st.start()
                st.wait()

    out_shape = jax.ShapeDtypeStruct((m, n), jnp.float32)
    return pl.pallas_call(
        body,
        out_shape=out_shape,
        in_specs=[pl.BlockSpec(memory_space=pl.ANY)],
        out_specs=pl.BlockSpec(memory_space=pl.ANY),
        scratch_shapes=[
            pltpu.VMEM((2, r_rows, n), jnp.float32),
            pltpu.VMEM((r_rows, n), jnp.float32),
            pltpu.VMEM((r_rows, n), jnp.float32),
            pltpu.SemaphoreType.DMA((2 * (N_DEV - 1),)),
            pltpu.SemaphoreType.DMA((2 * (N_DEV - 1),)),
            pltpu.SemaphoreType.DMA,
            pltpu.SemaphoreType.DMA,
            pltpu.SemaphoreType.REGULAR,
        ],
        compiler_params=pltpu.CompilerParams(collective_id=0),
    )(t)


# device time: 654245 ns/iter; 1.8257x vs baseline; 1.8257x over previous
import jax
import jax.numpy as jnp
from jax import lax
from jax.experimental import pallas as pl
from jax.experimental.pallas import tpu as pltpu

N_DEV = 4
NPASS = 2
N_HOP = 2 * (N_DEV - 1)


def _f(s):
    r = jnp.maximum(s, 0.0)
    return jnp.tanh(s) * s * s + r * r * r


def kernel(t):
    m, n = t.shape
    c = m // N_DEV
    r_rows = c // NPASS
    half = n // 2

    def body(t_ref, out_ref, comm_ref, acc_ref, local_ref,
             send_sems, recv_sems, load_sems, store_sems, hop_sem):
        my = lax.axis_index("i")
        left = (my + N_DEV - 1) % N_DEV
        right = (my + 1) % N_DEV
        to = (right, left)

        def cols(d):
            return pl.ds(d * half, half)

        def nbr_barrier(sem):
            for nbr in (left, right):
                pl.semaphore_signal(
                    sem, inc=1,
                    device_id=(nbr,), device_id_type=pl.DeviceIdType.MESH,
                )
            pl.semaphore_wait(sem, 2)

        nbr_barrier(pltpu.get_barrier_semaphore())

        for p in range(NPASS):
            def rows(chunk_idx):
                return pl.ds(chunk_idx * c + p * r_rows, r_rows)

            for d in range(2):
                cp = pltpu.make_async_copy(
                    t_ref.at[rows(my), cols(d)], acc_ref.at[d], load_sems.at[d]
                )
                cp.start()
            for d in range(2):
                pltpu.make_async_copy(
                    t_ref.at[rows(my), cols(d)], acc_ref.at[d], load_sems.at[d]
                ).wait()

            for s in range(N_DEV - 1):
                h = s
                if not (p == 0 and h == 0):
                    nbr_barrier(hop_sem)
                slot = h % 2
                rdmas = []
                for d in range(2):
                    rdma = pltpu.make_async_remote_copy(
                        src_ref=acc_ref.at[d],
                        dst_ref=comm_ref.at[d, slot],
                        send_sem=send_sems.at[d, h],
                        recv_sem=recv_sems.at[d, h],
                        device_id=(to[d],),
                        device_id_type=pl.DeviceIdType.MESH,
                    )
                    rdma.start()
                    rdmas.append(rdma)
                incoming = (
                    (my + N_DEV - s - 1) % N_DEV,
                    (my + s + 1) % N_DEV,
                )
                loads = []
                for d in range(2):
                    load = pltpu.make_async_copy(
                        t_ref.at[rows(incoming[d]), cols(d)],
                        local_ref.at[d], load_sems.at[d],
                    )
                    load.start()
                    loads.append(load)
                for d in range(2):
                    rdmas[d].wait()
                    loads[d].wait()
                    acc_ref[d] = comm_ref[d, slot] + local_ref[d]

            own = ((my + 1) % N_DEV, (my + N_DEV - 1) % N_DEV)
            for d in range(2):
                acc_ref[d] = _f(acc_ref[d])
                st = pltpu.make_async_copy(
                    acc_ref.at[d], out_ref.at[rows(own[d]), cols(d)],
                    store_sems.at[d],
                )
                st.start()
            for d in range(2):
                pltpu.make_async_copy(
                    acc_ref.at[d], out_ref.at[rows(own[d]), cols(d)],
                    store_sems.at[d],
                ).wait()

            for s in range(N_DEV - 1):
                h = N_DEV - 1 + s
                nbr_barrier(hop_sem)
                slot = h % 2
                rdmas = []
                for d in range(2):
                    src = acc_ref.at[d] if s == 0 else comm_ref.at[d, (h - 1) % 2]
                    rdma = pltpu.make_async_remote_copy(
                        src_ref=src,
                        dst_ref=comm_ref.at[d, slot],
                        send_sem=send_sems.at[d, h],
                        recv_sem=recv_sems.at[d, h],
                        device_id=(to[d],),
                        device_id_type=pl.DeviceIdType.MESH,
                    )
                    rdma.start()
                    rdmas.append(rdma)
                got = (
                    (my + N_DEV - s) % N_DEV,
                    (my + s) % N_DEV,
                )
                for d in range(2):
                    rdmas[d].wait()
                    st = pltpu.make_async_copy(
                        comm_ref.at[d, slot],
                        out_ref.at[rows(got[d]), cols(d)],
                        store_sems.at[d],
                    )
                    st.start()
                for d in range(2):
                    pltpu.make_async_copy(
                        comm_ref.at[d, slot],
                        out_ref.at[rows(got[d]), cols(d)],
                        store_sems.at[d],
                    ).wait()

    out_shape = jax.ShapeDtypeStruct((m, n), jnp.float32)
    return pl.pallas_call(
        body,
        out_shape=out_shape,
        in_specs=[pl.BlockSpec(memory_space=pl.ANY)],
        out_specs=pl.BlockSpec(memory_space=pl.ANY),
        scratch_shapes=[
            pltpu.VMEM((2, 2, r_rows, half), jnp.float32),
            pltpu.VMEM((2, r_rows, half), jnp.float32),
            pltpu.VMEM((2, r_rows, half), jnp.float32),
            pltpu.SemaphoreType.DMA((2, N_HOP)),
            pltpu.SemaphoreType.DMA((2, N_HOP)),
            pltpu.SemaphoreType.DMA((2,)),
            pltpu.SemaphoreType.DMA((2,)),
            pltpu.SemaphoreType.REGULAR,
        ],
        compiler_params=pltpu.CompilerParams(collective_id=0),
    )(t)


# device time: 615018 ns/iter; 1.9422x vs baseline; 1.0638x over previous
import jax
import jax.numpy as jnp
from jax import lax
from jax.experimental import pallas as pl
from jax.experimental.pallas import tpu as pltpu

N_DEV = 4
NPASS = 2
N_HOP = 2 * (N_DEV - 1)


def _f(s):
    r = jnp.maximum(s, 0.0)
    return jnp.tanh(s) * s * s + r * r * r


def kernel(t):
    m, n = t.shape
    c = m // N_DEV
    r_rows = c // NPASS
    half = n // 2

    def body(t_ref, out_ref, comm_ref, acc_ref, local_ref,
             send_sems, recv_sems, load_sems, store_sems, hop_sem):
        my = lax.axis_index("i")
        left = (my + N_DEV - 1) % N_DEV
        right = (my + 1) % N_DEV
        to = (right, left)

        def cols(d):
            return pl.ds(d * half, half)

        def rows(chunk_idx, q):
            return pl.ds(chunk_idx * c + q * r_rows, r_rows)

        def nbr_barrier(sem):
            for nbr in (left, right):
                pl.semaphore_signal(
                    sem, inc=1,
                    device_id=(nbr,), device_id_type=pl.DeviceIdType.MESH,
                )
            pl.semaphore_wait(sem, 2)

        nbr_barrier(pltpu.get_barrier_semaphore())

        for q in range(NPASS):
            seeds = [
                pltpu.make_async_copy(
                    t_ref.at[rows(my, q), cols(d)], acc_ref.at[q, d],
                    load_sems.at[d],
                )
                for d in range(2)
            ]
            for s in seeds:
                s.start()
            for s in seeds:
                s.wait()

        pending_stores = {}
        pending_sends = []

        def start_store(q, d, src, dst):
            if (q, d) in pending_stores:
                pending_stores[(q, d)].wait()
            st = pltpu.make_async_copy(src, dst, store_sems.at[q, d])
            st.start()
            pending_stores[(q, d)] = st

        for h in range(N_HOP):
            slot = h % 2
            for r in pending_sends:
                r.wait_send()
            pending_sends = []
            if h > 0:
                nbr_barrier(hop_sem)

            if h < N_DEV - 1:
                s = h
                incoming = ((my + N_DEV - s - 1) % N_DEV, (my + s + 1) % N_DEV)
                rdmas = [
                    [
                        pltpu.make_async_remote_copy(
                            src_ref=acc_ref.at[q, d],
                            dst_ref=comm_ref.at[q, d, slot],
                            send_sem=send_sems.at[q, d, slot],
                            recv_sem=recv_sems.at[q, d, slot],
                            device_id=(to[d],),
                            device_id_type=pl.DeviceIdType.MESH,
                        )
                        for d in range(2)
                    ]
                    for q in range(NPASS)
                ]
                for q in range(NPASS):
                    for d in range(2):
                        rdmas[q][d].start()
                for q in range(NPASS):
                    loads = [
                        pltpu.make_async_copy(
                            t_ref.at[rows(incoming[d], q), cols(d)],
                            local_ref.at[d], load_sems.at[d],
                        )
                        for d in range(2)
                    ]
                    for ld in loads:
                        ld.start()
                    for d in range(2):
                        rdmas[q][d].wait()
                        loads[d].wait()
                        acc_ref[q, d] = comm_ref[q, d, slot] + local_ref[d]
                        if s == N_DEV - 2:
                            own = (my + 1) % N_DEV if d == 0 else left
                            acc_ref[q, d] = _f(acc_ref[q, d])
                            start_store(
                                q, d, acc_ref.at[q, d],
                                out_ref.at[rows(own, q), cols(d)],
                            )
            else:
                s = h - (N_DEV - 1)
                got = ((my + N_DEV - s) % N_DEV, (my + s) % N_DEV)
                rdmas = [
                    [
                        pltpu.make_async_remote_copy(
                            src_ref=(
                                acc_ref.at[q, d] if s == 0
                                else comm_ref.at[q, d, (h - 1) % 2]
                            ),
                            dst_ref=comm_ref.at[q, d, slot],
                            send_sem=send_sems.at[q, d, slot],
                            recv_sem=recv_sems.at[q, d, slot],
                            device_id=(to[d],),
                            device_id_type=pl.DeviceIdType.MESH,
                        )
                        for d in range(2)
                    ]
                    for q in range(NPASS)
                ]
                for q in range(NPASS):
                    for d in range(2):
                        rdmas[q][d].start()
                for q in range(NPASS):
                    for d in range(2):
                        rdmas[q][d].wait_recv()
                        start_store(
                            q, d, comm_ref.at[q, d, slot],
                            out_ref.at[rows(got[d], q), cols(d)],
                        )
                        pending_sends.append(rdmas[q][d])

        for r in pending_sends:
            r.wait_send()
        for st in pending_stores.values():
            st.wait()

    out_shape = jax.ShapeDtypeStruct((m, n), jnp.float32)
    return pl.pallas_call(
        body,
        out_shape=out_shape,
        in_specs=[pl.BlockSpec(memory_space=pl.ANY)],
        out_specs=pl.BlockSpec(memory_space=pl.ANY),
        scratch_shapes=[
            pltpu.VMEM((NPASS, 2, 2, r_rows, half), jnp.float32),
            pltpu.VMEM((NPASS, 2, r_rows, half), jnp.float32),
            pltpu.VMEM((2, r_rows, half), jnp.float32),
            pltpu.SemaphoreType.DMA((NPASS, 2, 2)),
            pltpu.SemaphoreType.DMA((NPASS, 2, 2)),
            pltpu.SemaphoreType.DMA((2,)),
            pltpu.SemaphoreType.DMA((NPASS, 2)),
            pltpu.SemaphoreType.REGULAR,
        ],
        compiler_params=pltpu.CompilerParams(
            collective_id=0, vmem_limit_bytes=62 * 1024 * 1024
        ),
    )(t)


# device time: 596342 ns/iter; 2.0030x vs baseline; 1.0313x over previous
import jax
import jax.numpy as jnp
from jax import lax
from jax.experimental import pallas as pl
from jax.experimental.pallas import tpu as pltpu

N_DEV = 4
NPASS = 2
N_HOP = 2 * (N_DEV - 1)


def _f(s):
    r = jnp.maximum(s, 0.0)
    return jnp.tanh(s) * s * s + r * r * r


def kernel(t):
    m, n = t.shape
    c = m // N_DEV
    r_rows = c // NPASS
    half = n // 2

    def body(t_ref, out_ref, comm_ref, acc_ref, local_ref,
             send_sems, recv_sems, load_sems, store_sems, credit_sems):
        my = lax.axis_index("i")
        left = (my + N_DEV - 1) % N_DEV
        right = (my + 1) % N_DEV
        to = (right, left)
        frm = (left, right)

        def cols(d):
            return pl.ds(d * half, half)

        def rows(chunk_idx, q):
            return pl.ds(chunk_idx * c + q * r_rows, r_rows)

        def make_rdma(h, q, d, src):
            return pltpu.make_async_remote_copy(
                src_ref=src,
                dst_ref=comm_ref.at[q, d, h % 2],
                send_sem=send_sems.at[q, d, h % 2],
                recv_sem=recv_sems.at[q, d, h % 2],
                device_id=(to[d],),
                device_id_type=pl.DeviceIdType.MESH,
            )

        def make_load(chunk_idx, q, d):
            return pltpu.make_async_copy(
                t_ref.at[rows(chunk_idx, q), cols(d)],
                local_ref.at[d], load_sems.at[q, d],
            )

        def credit_wait(h):
            if h >= 2:
                pl.semaphore_wait(credit_sems.at[h % 2], 2)

        def credit_signal(slot):
            for d in range(2):
                pl.semaphore_signal(
                    credit_sems.at[slot], inc=1,
                    device_id=(frm[d],), device_id_type=pl.DeviceIdType.MESH,
                )

        pending_stores = {}

        def start_store(q, d, src, dst):
            if (q, d) in pending_stores:
                pending_stores[(q, d)].wait()
            st = pltpu.make_async_copy(src, dst, store_sems.at[q, d])
            st.start()
            pending_stores[(q, d)] = st

        seeds = [[
            pltpu.make_async_copy(
                t_ref.at[rows(my, q), cols(d)], acc_ref.at[q, d],
                load_sems.at[q, d],
            )
            for d in range(2)] for q in range(NPASS)
        ]
        for q in range(NPASS):
            for d in range(2):
                seeds[q][d].start()

        barrier_sem = pltpu.get_barrier_semaphore()
        for nbr in (left, right):
            pl.semaphore_signal(
                barrier_sem, inc=1,
                device_id=(nbr,), device_id_type=pl.DeviceIdType.MESH,
            )
        pl.semaphore_wait(barrier_sem, 2)

        rdmas = [[[None, None] for _ in range(NPASS)] for _ in range(N_HOP)]
        loads = [[[None, None] for _ in range(NPASS)] for _ in range(N_DEV - 1)]

        for q in range(NPASS):
            for d in range(2):
                seeds[q][d].wait()
            for d in range(2):
                rdmas[0][q][d] = make_rdma(0, q, d, acc_ref.at[q, d])
                rdmas[0][q][d].start()
        rs_incoming = lambda s: ((my + N_DEV - s - 1) % N_DEV,
                                 (my + s + 1) % N_DEV)
        for d in range(2):
            loads[0][0][d] = make_load(rs_incoming(0)[d], 0, d)
            loads[0][0][d].start()

        own = ((my + 1) % N_DEV, left)
        for h in range(N_DEV - 1):
            s = h
            last = s == N_DEV - 2
            for q in range(NPASS):
                for d in range(2):
                    rdmas[h][q][d].wait()
                    loads[h][q][d].wait()
                    acc_ref[q, d] = comm_ref[q, d, h % 2] + local_ref[d]
                    if last:
                        acc_ref[q, d] = _f(acc_ref[q, d])
                        start_store(q, d, acc_ref.at[q, d],
                                    out_ref.at[rows(own[d], q), cols(d)])
                if q == NPASS - 1:
                    credit_signal(h % 2)
                if q == 0:
                    credit_wait(h + 1)
                for d in range(2):
                    rdmas[h + 1][q][d] = make_rdma(h + 1, q, d,
                                                   acc_ref.at[q, d])
                    rdmas[h + 1][q][d].start()
                nh, nq = (h, 1) if q == 0 else (h + 1, 0)
                if nh < N_DEV - 1:
                    for d in range(2):
                        loads[nh][nq][d] = make_load(rs_incoming(nh)[d], nq, d)
                        loads[nh][nq][d].start()

        def ag_got(s):
            return ((my + N_DEV - s) % N_DEV, (my + s) % N_DEV)

        for q in range(NPASS):
            for d in range(2):
                rdmas[3][q][d].wait_recv()
                start_store(q, d, comm_ref.at[q, d, 1],
                            out_ref.at[rows(ag_got(0)[d], q), cols(d)])
            if q == 0:
                credit_wait(4)
            for d in range(2):
                rdmas[4][q][d] = make_rdma(4, q, d, comm_ref.at[q, d, 1])
                rdmas[4][q][d].start()

        for q in range(NPASS):
            for d in range(2):
                rdmas[3][q][d].wait_send()
        for q in range(NPASS):
            for d in range(2):
                rdmas[4][q][d].wait_recv()
                start_store(q, d, comm_ref.at[q, d, 0],
                            out_ref.at[rows(ag_got(1)[d], q), cols(d)])
        for q in range(NPASS):
            for d in range(2):
                rdmas[4][q][d].wait_send()
        credit_signal(1)
        credit_wait(5)
        for q in range(NPASS):
            for d in range(2):
                rdmas[5][q][d] = make_rdma(5, q, d, comm_ref.at[q, d, 0])
                rdmas[5][q][d].start()

        for q in range(NPASS):
            for d in range(2):
                rdmas[5][q][d].wait_recv()
                start_store(q, d, comm_ref.at[q, d, 1],
                            out_ref.at[rows(ag_got(2)[d], q), cols(d)])

        for q in range(NPASS):
            for d in range(2):
                rdmas[N_HOP - 1][q][d].wait_send()
        for st in pending_stores.values():
            st.wait()

    out_shape = jax.ShapeDtypeStruct((m, n), jnp.float32)
    return pl.pallas_call(
        body,
        out_shape=out_shape,
        in_specs=[pl.BlockSpec(memory_space=pl.ANY)],
        out_specs=pl.BlockSpec(memory_space=pl.ANY),
        scratch_shapes=[
            pltpu.VMEM((NPASS, 2, 2, r_rows, half), jnp.float32),
            pltpu.VMEM((NPASS, 2, r_rows, half), jnp.float32),
            pltpu.VMEM((2, r_rows, half), jnp.float32),
            pltpu.SemaphoreType.DMA((NPASS, 2, 2)),
            pltpu.SemaphoreType.DMA((NPASS, 2, 2)),
            pltpu.SemaphoreType.DMA((NPASS, 2)),
            pltpu.SemaphoreType.DMA((NPASS, 2)),
            pltpu.SemaphoreType.REGULAR((2,)),
        ],
        compiler_params=pltpu.CompilerParams(
            collective_id=0, vmem_limit_bytes=62 * 1024 * 1024
        ),
    )(t)
